# baseline (speedup 1.0000x reference)
import functools

import jax
import jax.numpy as jnp
from jax import lax
from jax.experimental import pallas as pl
from jax.experimental.pallas import tpu as pltpu
from jax.experimental.pallas import tpu_sc as plsc

_NUM_CORES = 2
_NUM_SUBCORES = 16
_NUM_WORKERS = _NUM_CORES * _NUM_SUBCORES
_LANES = 16

_BATCH = 64
_SEQ = 1024
_HIDDEN = 512
_NUM_ROWS = 32
_TOTAL = _BATCH * _SEQ
_BPW = _TOTAL // _NUM_WORKERS
_CHUNK = 64
_NCHUNK = _BPW // _CHUNK
_GROUPS = _CHUNK // _LANES


def _body(x_hbm, tab_hbm, out_hbm, idx_v, tab_v, rows0, rows1, sem0, sem1):
    wid = lax.axis_index("s") * _NUM_CORES + lax.axis_index("c")
    base = wid * _BPW
    pltpu.sync_copy(tab_hbm, tab_v)
    pltpu.sync_copy(x_hbm.at[pl.ds(base, _BPW)], idx_v)

    pos_lane = lax.iota(jnp.int32, _LANES) * _HIDDEN

    rows = (rows0, rows1)
    sems = (sem0, sem1)
    scats = [None, None]
    for g in range(_NCHUNK):
        s = g % 2
        if scats[s] is not None:
            scats[s].wait()
        rbuf = rows[s]
        for j in range(_GROUPS):
            idx16 = idx_v[pl.ds(g * _CHUNK + j * _LANES, _LANES)]
            addr0 = idx16 * _HIDDEN
            pos0 = pos_lane + j * _LANES * _HIDDEN

            @plsc.parallel_loop(0, _HIDDEN, unroll=8, carry=(addr0, pos0))
            def _cols(c, carry, rbuf=rbuf):
                addr, pos = carry
                v = plsc.load_gather(tab_v, [addr])
                plsc.store_scatter(rbuf, [pos], v)
                return (addr + 1, pos + 1)

        scats[s] = pltpu.async_copy(
            rbuf,
            out_hbm.at[pl.ds((base + g * _CHUNK) * _HIDDEN,
                             _CHUNK * _HIDDEN)],
            sems[s])
    scats[0].wait()
    scats[1].wait()


@jax.jit
def _lookup(x_flat, tab_flat):
    mesh = plsc.VectorSubcoreMesh(core_axis_name="c", subcore_axis_name="s")
    run = pl.kernel(
        _body,
        out_type=jax.ShapeDtypeStruct((_TOTAL * _HIDDEN,), jnp.float32),
        mesh=mesh,
        compiler_params=pltpu.CompilerParams(needs_layout_passes=False),
        scratch_types=[
            pltpu.VMEM((_BPW,), jnp.int32),
            pltpu.VMEM((_NUM_ROWS * _HIDDEN,), jnp.float32),
            pltpu.VMEM((_CHUNK * _HIDDEN,), jnp.float32),
            pltpu.VMEM((_CHUNK * _HIDDEN,), jnp.float32),
            pltpu.SemaphoreType.DMA,
            pltpu.SemaphoreType.DMA,
        ],
    )
    return run(x_flat, tab_flat)


def kernel(x, token_embed):
    out = _lookup(x.reshape(_TOTAL).astype(jnp.int32),
                  token_embed.reshape(_NUM_ROWS * _HIDDEN))
    return out.reshape(_BATCH, _SEQ, _HIDDEN)


# speedup vs baseline: 2.6935x; 2.6935x over previous
import functools

import jax
import jax.numpy as jnp
from jax import lax
from jax.experimental import pallas as pl
from jax.experimental.pallas import tpu as pltpu
from jax.experimental.pallas import tpu_sc as plsc

_NUM_CORES = 2
_NUM_SUBCORES = 16
_NUM_WORKERS = _NUM_CORES * _NUM_SUBCORES
_LANES = 16

_BATCH = 64
_SEQ = 1024
_HIDDEN = 512
_NUM_ROWS = 32
_TOTAL = _BATCH * _SEQ
_BPW = _TOTAL // _NUM_WORKERS
_CHUNK = 64
_NCHUNK = _BPW // _CHUNK
_GROUPS = _CHUNK // _LANES


def _body(x_hbm, tab_hbm, out_hbm, idx_v, tab_v, rows0, rows1, sem0, sem1):
    wid = lax.axis_index("s") * _NUM_CORES + lax.axis_index("c")
    base = wid * _BPW
    pltpu.sync_copy(tab_hbm, tab_v)
    pltpu.sync_copy(x_hbm.at[pl.ds(base, _BPW)], idx_v)

    lane = lax.iota(jnp.int32, _LANES)

    rows = (rows0, rows1)
    sems = (sem0, sem1)
    scats = [None, None]
    for g in range(_NCHUNK):
        s = g % 2
        if scats[s] is not None:
            scats[s].wait()
        rbuf = rows[s]
        goff = g * _CHUNK

        @plsc.parallel_loop(0, _CHUNK)
        def _row(r, rbuf=rbuf, goff=goff):
            vidx = jnp.full((_LANES,), goff + r, jnp.int32)
            irow = plsc.load_gather(idx_v, [vidx])
            addr0 = irow * _HIDDEN + lane
            pos0 = r * _HIDDEN

            @plsc.parallel_loop(0, _HIDDEN, step=_LANES, unroll=8,
                                carry=(addr0, pos0))
            def _cols(c, carry):
                addr, pos = carry
                v = plsc.load_gather(tab_v, [addr])
                rbuf[pl.ds(pos, _LANES)] = v
                return (addr + _LANES, pos + _LANES)

        scats[s] = pltpu.async_copy(
            rbuf,
            out_hbm.at[pl.ds((base + g * _CHUNK) * _HIDDEN,
                             _CHUNK * _HIDDEN)],
            sems[s])
    scats[0].wait()
    scats[1].wait()


@jax.jit
def _lookup(x_flat, tab_flat):
    mesh = plsc.VectorSubcoreMesh(core_axis_name="c", subcore_axis_name="s")
    run = pl.kernel(
        _body,
        out_type=jax.ShapeDtypeStruct((_TOTAL * _HIDDEN,), jnp.float32),
        mesh=mesh,
        compiler_params=pltpu.CompilerParams(needs_layout_passes=False),
        scratch_types=[
            pltpu.VMEM((_BPW,), jnp.int32),
            pltpu.VMEM((_NUM_ROWS * _HIDDEN,), jnp.float32),
            pltpu.VMEM((_CHUNK * _HIDDEN,), jnp.float32),
            pltpu.VMEM((_CHUNK * _HIDDEN,), jnp.float32),
            pltpu.SemaphoreType.DMA,
            pltpu.SemaphoreType.DMA,
        ],
    )
    return run(x_flat, tab_flat)


def kernel(x, token_embed):
    out = _lookup(x.reshape(_TOTAL).astype(jnp.int32),
                  token_embed.reshape(_NUM_ROWS * _HIDDEN))
    return out.reshape(_BATCH, _SEQ, _HIDDEN)


# speedup vs baseline: 3.0494x; 1.1321x over previous
import functools

import jax
import jax.numpy as jnp
from jax import lax
from jax.experimental import pallas as pl
from jax.experimental.pallas import tpu as pltpu
from jax.experimental.pallas import tpu_sc as plsc

_NUM_CORES = 2
_NUM_SUBCORES = 16
_NUM_WORKERS = _NUM_CORES * _NUM_SUBCORES
_LANES = 16

_BATCH = 64
_SEQ = 1024
_HIDDEN = 512
_NUM_ROWS = 32
_TOTAL = _BATCH * _SEQ
_BPW = _TOTAL // _NUM_WORKERS
_CHUNK = 64
_NCHUNK = _BPW // _CHUNK
_GROUPS = _CHUNK // _LANES


def _body(x_hbm, tab_hbm, out_hbm, idx_v, tab_v, rows0, rows1, sem0, sem1):
    wid = lax.axis_index("s") * _NUM_CORES + lax.axis_index("c")
    base = wid * _BPW
    pltpu.sync_copy(tab_hbm, tab_v)
    pltpu.sync_copy(x_hbm.at[pl.ds(base, _BPW)], idx_v)

    lane = lax.iota(jnp.int32, _LANES)

    rows = (rows0, rows1)
    sems = (sem0, sem1)

    def build(goff, rbuf):
        @plsc.parallel_loop(0, _CHUNK)
        def _row(r):
            vidx = jnp.full((_LANES,), goff + r, jnp.int32)
            irow = plsc.load_gather(idx_v, [vidx])
            addr0 = irow * _HIDDEN + lane
            pos0 = r * _HIDDEN
            for cb in range(_HIDDEN // _LANES):
                v = plsc.load_gather(tab_v, [addr0 + cb * _LANES])
                rbuf[pl.ds(pos0 + cb * _LANES, _LANES)] = v

    def fire(g, rbuf, sem):
        return pltpu.async_copy(
            rbuf,
            out_hbm.at[pl.ds((base + g * _CHUNK) * _HIDDEN,
                             _CHUNK * _HIDDEN)],
            sem)

    def drain(sem):
        pltpu.make_async_copy(
            rows0, out_hbm.at[pl.ds(base * _HIDDEN, _CHUNK * _HIDDEN)],
            sem).wait()

    for s in (0, 1):
        build(jnp.int32(s * _CHUNK), rows[s])
        fire(jnp.int32(s), rows[s], sems[s])

    def _pair(i, carry):
        for s in (0, 1):
            g = 2 * i + s
            drain(sems[s])
            build(g * _CHUNK, rows[s])
            fire(g, rows[s], sems[s])
        return carry

    lax.fori_loop(1, _NCHUNK // 2, _pair, jnp.int32(0))
    drain(sems[0])
    drain(sems[1])


@jax.jit
def _lookup(x_flat, tab_flat):
    mesh = plsc.VectorSubcoreMesh(core_axis_name="c", subcore_axis_name="s")
    run = pl.kernel(
        _body,
        out_type=jax.ShapeDtypeStruct((_TOTAL * _HIDDEN,), jnp.float32),
        mesh=mesh,
        compiler_params=pltpu.CompilerParams(needs_layout_passes=False),
        scratch_types=[
            pltpu.VMEM((_BPW,), jnp.int32),
            pltpu.VMEM((_NUM_ROWS * _HIDDEN,), jnp.float32),
            pltpu.VMEM((_CHUNK * _HIDDEN,), jnp.float32),
            pltpu.VMEM((_CHUNK * _HIDDEN,), jnp.float32),
            pltpu.SemaphoreType.DMA,
            pltpu.SemaphoreType.DMA,
        ],
    )
    return run(x_flat, tab_flat)


def kernel(x, token_embed):
    out = _lookup(x.reshape(_TOTAL).astype(jnp.int32),
                  token_embed.reshape(_NUM_ROWS * _HIDDEN))
    return out.reshape(_BATCH, _SEQ, _HIDDEN)


# speedup vs baseline: 9.0840x; 2.9789x over previous
import functools

import jax
import jax.numpy as jnp
from jax import lax
from jax.experimental import pallas as pl
from jax.experimental.pallas import tpu as pltpu
from jax.experimental.pallas import tpu_sc as plsc

_NUM_CORES = 2
_NUM_SUBCORES = 16
_NUM_WORKERS = _NUM_CORES * _NUM_SUBCORES
_LANES = 16

_BATCH = 64
_SEQ = 1024
_HIDDEN = 512
_NUM_ROWS = 32
_TOTAL = _BATCH * _SEQ
_BPW = _TOTAL // _NUM_WORKERS
_CHUNK = 64
_NCHUNK = _BPW // _CHUNK
_GROUPS = _CHUNK // _LANES


def _body(x_hbm, tab_hbm, out_hbm, idx_v, tab_v, rows0, rows1, sem0, sem1):
    wid = lax.axis_index("s") * _NUM_CORES + lax.axis_index("c")
    base = wid * _BPW
    pltpu.sync_copy(tab_hbm, tab_v)
    pltpu.sync_copy(x_hbm.at[pl.ds(base, _BPW)], idx_v)

    lane = lax.iota(jnp.int32, _LANES)

    rows = (rows0, rows1)
    sems = (sem0, sem1)

    def build(goff, rbuf):
        @plsc.parallel_loop(0, _CHUNK)
        def _row(r):
            vidx = jnp.full((_LANES,), goff + r, jnp.int32)
            irow = plsc.load_gather(idx_v, [vidx])
            addr0 = irow * _HIDDEN + lane
            for cb in range(_HIDDEN // _LANES):
                v = plsc.load_gather(tab_v, [addr0 + cb * _LANES])
                rbuf[r, pl.ds(cb * _LANES, _LANES)] = v

    def fire(g, rbuf, sem):
        return pltpu.async_copy(
            rbuf,
            out_hbm.at[pl.ds(base + g * _CHUNK, _CHUNK)],
            sem)

    def drain(sem):
        pltpu.make_async_copy(
            rows0, out_hbm.at[pl.ds(base, _CHUNK)], sem).wait()

    for s in (0, 1):
        build(jnp.int32(s * _CHUNK), rows[s])
        fire(jnp.int32(s), rows[s], sems[s])

    def _pair(i, carry):
        for s in (0, 1):
            g = 2 * i + s
            drain(sems[s])
            build(g * _CHUNK, rows[s])
            fire(g, rows[s], sems[s])
        return carry

    lax.fori_loop(1, _NCHUNK // 2, _pair, jnp.int32(0))
    drain(sems[0])
    drain(sems[1])


@jax.jit
def _lookup(x_flat, tab_flat):
    mesh = plsc.VectorSubcoreMesh(core_axis_name="c", subcore_axis_name="s")
    run = pl.kernel(
        _body,
        out_type=jax.ShapeDtypeStruct((_TOTAL, _HIDDEN), jnp.float32),
        mesh=mesh,
        compiler_params=pltpu.CompilerParams(needs_layout_passes=False),
        scratch_types=[
            pltpu.VMEM((_BPW,), jnp.int32),
            pltpu.VMEM((_NUM_ROWS * _HIDDEN,), jnp.float32),
            pltpu.VMEM((_CHUNK, _HIDDEN), jnp.float32),
            pltpu.VMEM((_CHUNK, _HIDDEN), jnp.float32),
            pltpu.SemaphoreType.DMA,
            pltpu.SemaphoreType.DMA,
        ],
    )
    return run(x_flat, tab_flat)


def kernel(x, token_embed):
    out = _lookup(x.reshape(_TOTAL).astype(jnp.int32),
                  token_embed.reshape(_NUM_ROWS * _HIDDEN))
    return out.reshape(_BATCH, _SEQ, _HIDDEN)
